# baseline (speedup 1.0000x reference)
import jax
import jax.numpy as jnp
from jax.experimental import pallas as pl


_BATCH_BLOCK = 144


def _pos_add_kernel(x_ref, pos_ref, o_ref):
    o_ref[...] = x_ref[...] + pos_ref[...]


def kernel(x, position_embeddings):
    batch, seq_len, d_model = x.shape
    pos = position_embeddings[:seq_len]
    bb = _BATCH_BLOCK
    grid = ((batch + bb - 1) // bb,)
    return pl.pallas_call(
        _pos_add_kernel,
        grid=grid,
        in_specs=[
            pl.BlockSpec((bb, seq_len, d_model), lambda i: (i, 0, 0)),
            pl.BlockSpec((seq_len, d_model), lambda i: (0, 0)),
        ],
        out_specs=pl.BlockSpec((bb, seq_len, d_model), lambda i: (i, 0, 0)),
        out_shape=jax.ShapeDtypeStruct((batch, seq_len, d_model), x.dtype),
    )(x, pos)
